# baseline (speedup 1.0000x reference)
import functools

import jax
import jax.numpy as jnp
from jax import lax
from jax.experimental import pallas as pl
from jax.experimental.pallas import tpu as pltpu
from jax.experimental.pallas import tpu_sc as plsc

_LANES = 128
_NW = 32


def _make_sc_gather(BT, D, CH, NCH):
    n_per_w = BT // _NW
    n_iter = n_per_w // CH
    idx_rows_per_w = n_per_w // _LANES
    mesh = plsc.VectorSubcoreMesh(core_axis_name="c", subcore_axis_name="s")

    @functools.partial(
        pl.kernel,
        mesh=mesh,
        out_type=jax.ShapeDtypeStruct((BT, D), jnp.float32),
        scratch_types=[
            pltpu.VMEM((NCH, _LANES), jnp.int32),
            pltpu.VMEM((CH, D), jnp.float32),
            pltpu.SemaphoreType.DMA,
        ],
        compiler_params=pltpu.CompilerParams(use_tc_tiling_on_sc=False),
    )
    def k(table_hbm, idx_hbm, out_hbm, idx_v, rows_v, sem):
        wid = lax.axis_index("s") * 2 + lax.axis_index("c")
        row0 = wid * n_per_w
        irow0 = wid * idx_rows_per_w

        def body(i, carry):
            pltpu.sync_copy(idx_hbm.at[pl.ds(irow0 + i * NCH, NCH)], idx_v)
            copies = [
                pltpu.async_copy(
                    table_hbm.at[idx_v.at[j]],
                    rows_v.at[pl.ds(j * _LANES, _LANES)],
                    sem,
                )
                for j in range(NCH)
            ]
            for c in copies:
                c.wait()
            pltpu.sync_copy(rows_v, out_hbm.at[pl.ds(row0 + i * CH, CH)])
            return carry

        lax.fori_loop(0, n_iter, body, 0)

    return k


def kernel(x, table):
    B, L = x.shape
    V, D = table.shape
    BT = B * L
    idx = x.reshape(BT // _LANES, _LANES).astype(jnp.int32)
    out = _make_sc_gather(BT, D, CH=1024, NCH=8)(table, idx)
    return out.reshape(B, L, D)


# speedup vs baseline: 1.0194x; 1.0194x over previous
import functools

import jax
import jax.numpy as jnp
from jax import lax
from jax.experimental import pallas as pl
from jax.experimental.pallas import tpu as pltpu
from jax.experimental.pallas import tpu_sc as plsc

_LANES = 128
_NW = 32


def _make_sc_gather(BT, D, CH, NCH):
    n_per_w = BT // _NW
    n_pair = n_per_w // (2 * CH)
    idx_rows_per_w = n_per_w // _LANES
    mesh = plsc.VectorSubcoreMesh(core_axis_name="c", subcore_axis_name="s")

    @functools.partial(
        pl.kernel,
        mesh=mesh,
        out_type=jax.ShapeDtypeStruct((BT, D), jnp.float32),
        scratch_types=[
            pltpu.VMEM((2 * NCH, _LANES), jnp.int32),
            pltpu.VMEM((CH, D), jnp.float32),
            pltpu.VMEM((CH, D), jnp.float32),
            pltpu.SemaphoreType.DMA,
            pltpu.SemaphoreType.DMA,
            pltpu.SemaphoreType.DMA,
        ],
        compiler_params=pltpu.CompilerParams(use_tc_tiling_on_sc=False),
    )
    def k(table_hbm, idx_hbm, out_hbm, idx_v, rows_a, rows_b, sem_g,
          sem_sa, sem_sb):
        wid = lax.axis_index("s") * 2 + lax.axis_index("c")
        row0 = wid * n_per_w
        irow0 = wid * idx_rows_per_w

        def gather(half, rows_v):
            return [
                pltpu.async_copy(
                    table_hbm.at[idx_v.at[half * NCH + j]],
                    rows_v.at[pl.ds(j * _LANES, _LANES)],
                    sem_g,
                )
                for j in range(NCH)
            ]

        def drain_store(rows_v, sem):
            pltpu.make_async_copy(out_hbm.at[pl.ds(0, CH)], rows_v, sem).wait()

        def body(i, carry):
            pltpu.sync_copy(
                idx_hbm.at[pl.ds(irow0 + i * 2 * NCH, 2 * NCH)], idx_v)

            @pl.when(i > 0)
            def _():
                drain_store(rows_a, sem_sa)

            ga = gather(0, rows_a)

            @pl.when(i > 0)
            def _():
                drain_store(rows_b, sem_sb)

            for c in ga:
                c.wait()
            pltpu.async_copy(
                rows_a, out_hbm.at[pl.ds(row0 + i * 2 * CH, CH)], sem_sa)

            gb = gather(1, rows_b)
            for c in gb:
                c.wait()
            pltpu.async_copy(
                rows_b, out_hbm.at[pl.ds(row0 + i * 2 * CH + CH, CH)], sem_sb)
            return carry

        lax.fori_loop(0, n_pair, body, 0)
        drain_store(rows_a, sem_sa)
        drain_store(rows_b, sem_sb)

    return k


def kernel(x, table):
    B, L = x.shape
    V, D = table.shape
    BT = B * L
    idx = x.reshape(BT // _LANES, _LANES).astype(jnp.int32)
    out = _make_sc_gather(BT, D, CH=512, NCH=4)(table, idx)
    return out.reshape(B, L, D)


# speedup vs baseline: 1.0227x; 1.0032x over previous
import functools

import jax
import jax.numpy as jnp
from jax import lax
from jax.experimental import pallas as pl
from jax.experimental.pallas import tpu as pltpu
from jax.experimental.pallas import tpu_sc as plsc

_NW = 32
_LL = 100


def _make_sc_gather(B, L, D, CHB):
    nl = L // _LL
    b_per_w = B // _NW
    n_pair = b_per_w // (2 * CHB)
    nlc = CHB * nl
    mesh = plsc.VectorSubcoreMesh(core_axis_name="c", subcore_axis_name="s")

    @functools.partial(
        pl.kernel,
        mesh=mesh,
        out_type=jax.ShapeDtypeStruct((B, L, D), jnp.float32),
        scratch_types=[
            pltpu.VMEM((2 * nlc, _LL), jnp.int32),
            pltpu.VMEM((CHB, L, D), jnp.float32),
            pltpu.VMEM((CHB, L, D), jnp.float32),
            pltpu.SemaphoreType.DMA,
            pltpu.SemaphoreType.DMA,
            pltpu.SemaphoreType.DMA,
        ],
        compiler_params=pltpu.CompilerParams(use_tc_tiling_on_sc=False),
    )
    def k(table_hbm, idx_hbm, out_hbm, idx_v, rows_a, rows_b, sem_g,
          sem_sa, sem_sb):
        wid = lax.axis_index("s") * 2 + lax.axis_index("c")
        b0w = wid * b_per_w

        def gather(half, rows_v):
            return [
                pltpu.async_copy(
                    table_hbm.at[idx_v.at[half * nlc + j]],
                    rows_v.at[j // nl, pl.ds((j % nl) * _LL, _LL)],
                    sem_g,
                )
                for j in range(nlc)
            ]

        def drain_store(rows_v, sem):
            pltpu.make_async_copy(out_hbm.at[pl.ds(0, CHB)], rows_v, sem).wait()

        def body(i, carry):
            pltpu.sync_copy(
                idx_hbm.at[pl.ds((b0w + i * 2 * CHB) * nl, 2 * nlc)], idx_v)

            @pl.when(i > 0)
            def _():
                drain_store(rows_a, sem_sa)

            ga = gather(0, rows_a)

            @pl.when(i > 0)
            def _():
                drain_store(rows_b, sem_sb)

            for c in ga:
                c.wait()
            pltpu.async_copy(
                rows_a, out_hbm.at[pl.ds(b0w + i * 2 * CHB, CHB)], sem_sa)

            gb = gather(1, rows_b)
            for c in gb:
                c.wait()
            pltpu.async_copy(
                rows_b, out_hbm.at[pl.ds(b0w + i * 2 * CHB + CHB, CHB)],
                sem_sb)
            return carry

        lax.fori_loop(0, n_pair, body, 0)
        drain_store(rows_a, sem_sa)
        drain_store(rows_b, sem_sb)

    return k


def kernel(x, table):
    B, L = x.shape
    V, D = table.shape
    idx = x.reshape(B * L // _LL, _LL).astype(jnp.int32)
    return _make_sc_gather(B, L, D, CHB=4)(table, idx)
